# baseline (speedup 1.0000x reference)
import functools

import jax
import jax.numpy as jnp
from jax import lax
from jax.experimental import pallas as pl
from jax.experimental.pallas import tpu as pltpu
from jax.experimental.pallas import tpu_sc as plsc

EMB = 64
LANES = 16
CHUNK = 128


def _scores_body(ent_hbm, rel_hbm, h_hbm, r_hbm, t_hbm, out_hbm,
                 idx0, idx1, col0, col1, h0, r0, t0, h1, r1, t1, out_v,
                 sem0, sem1, *, n_per_worker):
    nc = 2
    wid = lax.axis_index("s") * nc + lax.axis_index("c")
    lane = lax.broadcasted_iota(jnp.int32, (LANES,), 0)
    dnums = lax.GatherDimensionNumbers(
        offset_dims=(), collapsed_slice_dims=(0,), start_index_map=(0,))
    bufs = ((idx0, col0, h0, r0, t0, sem0), (idx1, col1, h1, r1, t1, sem1))
    n_chunks = n_per_worker // CHUNK

    def fold(x, d):
        shuf = lax.gather(x, (lane ^ d)[:, None], dnums, (1,),
                          mode=lax.GatherScatterMode.PROMISE_IN_BOUNDS)
        return x + shuf

    def stage_and_fire(chunk, slot):
        idx_v, col_v, h_rows, r_rows, t_rows, sem = bufs[slot]
        base = wid * n_per_worker + chunk * CHUNK
        pltpu.sync_copy(h_hbm.at[pl.ds(base, CHUNK)],
                        idx_v.at[pl.ds(0, CHUNK)])
        pltpu.sync_copy(r_hbm.at[pl.ds(base, CHUNK)],
                        idx_v.at[pl.ds(CHUNK, CHUNK)])
        pltpu.sync_copy(t_hbm.at[pl.ds(base, CHUNK)],
                        idx_v.at[pl.ds(2 * CHUNK, CHUNK)])

        def fire(g, _):
            hvec = idx_v[pl.ds(g * LANES, LANES)]
            rvec = idx_v[pl.ds(CHUNK + g * LANES, LANES)]
            tvec = idx_v[pl.ds(2 * CHUNK + g * LANES, LANES)]
            col_v[pl.ds(g * LANES, LANES)] = (hvec & 1) * EMB
            col_v[pl.ds(CHUNK + g * LANES, LANES)] = (rvec & 1) * EMB
            col_v[pl.ds(2 * CHUNK + g * LANES, LANES)] = (tvec & 1) * EMB
            hrow = lax.shift_right_logical(hvec, 1)
            rrow = lax.shift_right_logical(rvec, 1)
            trow = lax.shift_right_logical(tvec, 1)
            for i in range(LANES):
                j = g * LANES + i
                pltpu.async_copy(ent_hbm.at[hrow[i]], h_rows.at[j], sem)
                pltpu.async_copy(rel_hbm.at[rrow[i]], r_rows.at[j], sem)
                pltpu.async_copy(ent_hbm.at[trow[i]], t_rows.at[j], sem)
            return 0

        lax.fori_loop(0, CHUNK // LANES, fire, 0)

    def drain(slot):
        _, _, h_rows, r_rows, t_rows, sem = bufs[slot]
        pltpu.make_async_copy(ent_hbm.at[pl.ds(0, CHUNK)], h_rows, sem).wait()
        pltpu.make_async_copy(ent_hbm.at[pl.ds(0, CHUNK)], r_rows, sem).wait()
        pltpu.make_async_copy(ent_hbm.at[pl.ds(0, CHUNK)], t_rows, sem).wait()

    def compute(chunk, slot):
        _, col_v, h_rows, r_rows, t_rows, _ = bufs[slot]
        base = wid * n_per_worker + chunk * CHUNK

        def grp(g, _):
            hcol = col_v[pl.ds(g * LANES, LANES)]
            rcol = col_v[pl.ds(CHUNK + g * LANES, LANES)]
            tcol = col_v[pl.ds(2 * CHUNK + g * LANES, LANES)]
            parts = []
            for i in range(LANES):
                idx = g * LANES + i
                ho = pl.multiple_of(hcol[i], EMB)
                ro = pl.multiple_of(rcol[i], EMB)
                to = pl.multiple_of(tcol[i], EMB)
                p = (h_rows[idx, pl.ds(ho, LANES)]
                     * r_rows[idx, pl.ds(ro, LANES)]
                     * t_rows[idx, pl.ds(to, LANES)])
                for k in range(1, EMB // LANES):
                    p = p + (h_rows[idx, pl.ds(ho + k * LANES, LANES)]
                             * r_rows[idx, pl.ds(ro + k * LANES, LANES)]
                             * t_rows[idx, pl.ds(to + k * LANES, LANES)])
                parts.append(p)
            d = 1
            while len(parts) > 1:
                sel = (lane & d) == 0
                parts = [jnp.where(sel, fold(a, d), fold(b, d))
                         for a, b in zip(parts[0::2], parts[1::2])]
                d *= 2
            out_v[pl.ds(g * LANES, LANES)] = parts[0]
            return 0

        lax.fori_loop(0, CHUNK // LANES, grp, 0)
        pltpu.sync_copy(out_v, out_hbm.at[pl.ds(base, CHUNK)])

    assert n_chunks % 2 == 0
    stage_and_fire(0, 0)

    def chunk_pair(c2, _):
        c0 = c2 * 2
        drain(0)
        stage_and_fire(c0 + 1, 1)
        compute(c0, 0)
        drain(1)

        @pl.when(c2 + 1 < n_chunks // 2)
        def _():
            stage_and_fire(c0 + 2, 0)

        compute(c0 + 1, 1)
        return 0

    lax.fori_loop(0, n_chunks // 2, chunk_pair, 0)


def _make_scores(total):
    info = plsc.get_sparse_core_info()
    nw = info.num_cores * info.num_subcores
    assert total % (nw * CHUNK) == 0
    n_per_worker = total // nw
    mesh = plsc.VectorSubcoreMesh(core_axis_name="c", subcore_axis_name="s")

    return pl.kernel(
        functools.partial(_scores_body, n_per_worker=n_per_worker),
        mesh=mesh,
        out_type=jax.ShapeDtypeStruct((total,), jnp.float32),
        scratch_types=[
            pltpu.VMEM((3 * CHUNK,), jnp.int32),
            pltpu.VMEM((3 * CHUNK,), jnp.int32),
            pltpu.VMEM((3 * CHUNK,), jnp.int32),
            pltpu.VMEM((3 * CHUNK,), jnp.int32),
            pltpu.VMEM((CHUNK, 2 * EMB), jnp.float32),
            pltpu.VMEM((CHUNK, 2 * EMB), jnp.float32),
            pltpu.VMEM((CHUNK, 2 * EMB), jnp.float32),
            pltpu.VMEM((CHUNK, 2 * EMB), jnp.float32),
            pltpu.VMEM((CHUNK, 2 * EMB), jnp.float32),
            pltpu.VMEM((CHUNK, 2 * EMB), jnp.float32),
            pltpu.VMEM((CHUNK,), jnp.float32),
            pltpu.SemaphoreType.DMA,
            pltpu.SemaphoreType.DMA,
        ],
    )


def kernel(entity_emb, relation_emb, pos_h, pos_r, pos_t, neg_h, neg_r, neg_t):
    batch = pos_h.shape[0]
    h = jnp.concatenate([pos_h, neg_h]).astype(jnp.int32)
    r = jnp.concatenate([pos_r, neg_r]).astype(jnp.int32)
    t = jnp.concatenate([pos_t, neg_t]).astype(jnp.int32)
    ent2 = jnp.reshape(entity_emb, (-1, 2 * EMB))
    rel2 = jnp.reshape(relation_emb, (-1, 2 * EMB))
    scores = _make_scores(2 * batch)(ent2, rel2, h, r, t)
    return scores[:batch], scores[batch:]


# speedup vs baseline: 1.6250x; 1.6250x over previous
import functools

import jax
import jax.numpy as jnp
from jax import lax
from jax.experimental import pallas as pl
from jax.experimental.pallas import tpu as pltpu
from jax.experimental.pallas import tpu_sc as plsc

EMB = 64
LANES = 16
CHUNK = 128


def _scores_body(ent_hbm, rel_hbm, h_hbm, r_hbm, t_hbm, out_hbm,
                 idx0, idx1, h0, r0, t0, h1, r1, t1, out_v, sem0, sem1,
                 *, n_per_worker):
    nc = 2
    wid = lax.axis_index("s") * nc + lax.axis_index("c")
    lane = lax.broadcasted_iota(jnp.int32, (LANES,), 0)
    dnums = lax.GatherDimensionNumbers(
        offset_dims=(), collapsed_slice_dims=(0,), start_index_map=(0,))
    bufs = ((idx0, h0, r0, t0, sem0), (idx1, h1, r1, t1, sem1))
    n_chunks = n_per_worker // CHUNK

    def fold(x, d):
        shuf = lax.gather(x, (lane ^ d)[:, None], dnums, (1,),
                          mode=lax.GatherScatterMode.PROMISE_IN_BOUNDS)
        return x + shuf

    def stage_and_fire(chunk, slot):
        idx_v, h_rows, r_rows, t_rows, sem = bufs[slot]
        base = wid * n_per_worker + chunk * CHUNK
        pltpu.sync_copy(h_hbm.at[pl.ds(base, CHUNK)],
                        idx_v.at[pl.ds(0, CHUNK)])
        pltpu.sync_copy(r_hbm.at[pl.ds(base, CHUNK)],
                        idx_v.at[pl.ds(CHUNK, CHUNK)])
        pltpu.sync_copy(t_hbm.at[pl.ds(base, CHUNK)],
                        idx_v.at[pl.ds(2 * CHUNK, CHUNK)])

        def fire(g, _):
            hvec = idx_v[pl.ds(g * LANES, LANES)]
            rvec = idx_v[pl.ds(CHUNK + g * LANES, LANES)]
            tvec = idx_v[pl.ds(2 * CHUNK + g * LANES, LANES)]
            for i in range(LANES):
                j = g * LANES + i
                pltpu.async_copy(ent_hbm.at[hvec[i]], h_rows.at[j], sem)
                pltpu.async_copy(rel_hbm.at[rvec[i]], r_rows.at[j], sem)
                pltpu.async_copy(ent_hbm.at[tvec[i]], t_rows.at[j], sem)
            return 0

        lax.fori_loop(0, CHUNK // LANES, fire, 0)

    def drain(slot):
        _, h_rows, r_rows, t_rows, sem = bufs[slot]
        pltpu.make_async_copy(ent_hbm.at[pl.ds(0, CHUNK)], h_rows, sem).wait()
        pltpu.make_async_copy(ent_hbm.at[pl.ds(0, CHUNK)], r_rows, sem).wait()
        pltpu.make_async_copy(ent_hbm.at[pl.ds(0, CHUNK)], t_rows, sem).wait()

    def compute(chunk, slot):
        _, h_rows, r_rows, t_rows, _ = bufs[slot]
        base = wid * n_per_worker + chunk * CHUNK

        def grp(g, _):
            parts = []
            for i in range(LANES):
                idx = g * LANES + i
                p = (h_rows[idx, pl.ds(0, LANES)]
                     * r_rows[idx, pl.ds(0, LANES)]
                     * t_rows[idx, pl.ds(0, LANES)])
                for k in range(1, EMB // LANES):
                    p = p + (h_rows[idx, pl.ds(k * LANES, LANES)]
                             * r_rows[idx, pl.ds(k * LANES, LANES)]
                             * t_rows[idx, pl.ds(k * LANES, LANES)])
                parts.append(p)
            d = 1
            while len(parts) > 1:
                sel = (lane & d) == 0
                parts = [jnp.where(sel, fold(a, d), fold(b, d))
                         for a, b in zip(parts[0::2], parts[1::2])]
                d *= 2
            out_v[pl.ds(g * LANES, LANES)] = parts[0]
            return 0

        lax.fori_loop(0, CHUNK // LANES, grp, 0)
        pltpu.sync_copy(out_v, out_hbm.at[pl.ds(base, CHUNK)])

    assert n_chunks % 2 == 0
    stage_and_fire(0, 0)

    def chunk_pair(c2, _):
        c0 = c2 * 2
        drain(0)
        stage_and_fire(c0 + 1, 1)
        compute(c0, 0)
        drain(1)

        @pl.when(c2 + 1 < n_chunks // 2)
        def _():
            stage_and_fire(c0 + 2, 0)

        compute(c0 + 1, 1)
        return 0

    lax.fori_loop(0, n_chunks // 2, chunk_pair, 0)


def _make_scores(total):
    info = plsc.get_sparse_core_info()
    nw = info.num_cores * info.num_subcores
    assert total % (nw * CHUNK) == 0
    n_per_worker = total // nw
    mesh = plsc.VectorSubcoreMesh(core_axis_name="c", subcore_axis_name="s")

    return pl.kernel(
        functools.partial(_scores_body, n_per_worker=n_per_worker),
        mesh=mesh,
        out_type=jax.ShapeDtypeStruct((total,), jnp.float32),
        scratch_types=[
            pltpu.VMEM((3 * CHUNK,), jnp.int32),
            pltpu.VMEM((3 * CHUNK,), jnp.int32),
            pltpu.VMEM((CHUNK, EMB), jnp.float32),
            pltpu.VMEM((CHUNK, EMB), jnp.float32),
            pltpu.VMEM((CHUNK, EMB), jnp.float32),
            pltpu.VMEM((CHUNK, EMB), jnp.float32),
            pltpu.VMEM((CHUNK, EMB), jnp.float32),
            pltpu.VMEM((CHUNK, EMB), jnp.float32),
            pltpu.VMEM((CHUNK,), jnp.float32),
            pltpu.SemaphoreType.DMA,
            pltpu.SemaphoreType.DMA,
        ],
    )


def kernel(entity_emb, relation_emb, pos_h, pos_r, pos_t, neg_h, neg_r, neg_t):
    batch = pos_h.shape[0]
    h = jnp.concatenate([pos_h, neg_h]).astype(jnp.int32)
    r = jnp.concatenate([pos_r, neg_r]).astype(jnp.int32)
    t = jnp.concatenate([pos_t, neg_t]).astype(jnp.int32)
    scores = _make_scores(2 * batch)(entity_emb, relation_emb, h, r, t)
    return scores[:batch], scores[batch:]
